# baseline (speedup 1.0000x reference)
import functools

import jax
import jax.numpy as jnp
from jax import lax
from jax.experimental import pallas as pl
from jax.experimental.pallas import tpu as pltpu
from jax.experimental.pallas import tpu_sc as plsc

NUM_SC = 2
NUM_TEC = 16
N_TILES = NUM_SC * NUM_TEC
CH = 80
NB = 5


def _xa_body(x_ref, w_ref, o_ref):
    o_ref[...] = jnp.dot(x_ref[...], w_ref[...], preferred_element_type=jnp.float32)


def _msg_body(gx_ref, ea_ref, ts_ref, We1_ref, be1_ref, We2_ref, be2_ref,
              Wt_ref, bt_ref, Wn1b_ref, bn1_ref, Wn2_ref, bn2_ref, msg_ref):
    h = jnp.maximum(
        jnp.dot(ea_ref[...], We1_ref[...], preferred_element_type=jnp.float32)
        + be1_ref[...], 0.0)
    ef = jnp.dot(h, We2_ref[...], preferred_element_type=jnp.float32) + be2_ref[...]
    z = ts_ref[...] * Wt_ref[...] + bt_ref[...]
    ef = ef * (1.0 / (1.0 + jnp.exp(-z)))
    pre = (gx_ref[...]
           + jnp.dot(ef, Wn1b_ref[...], preferred_element_type=jnp.float32)
           + bn1_ref[...])
    msg_ref[...] = (jnp.dot(jnp.maximum(pre, 0.0), Wn2_ref[...],
                            preferred_element_type=jnp.float32) + bn2_ref[...])


def _combine_body(*refs):
    o_ref = refs[-1]
    x_ref = refs[-2]
    tot = x_ref[...]
    for acc_ref in refs[:-2]:
        tot = tot + acc_ref[0] + acc_ref[1]
    o_ref[...] = tot


def _make_gather(N, D, E):
    per_tile = E // (N_TILES * CH)
    mesh = plsc.VectorSubcoreMesh(core_axis_name="c", subcore_axis_name="s")

    @functools.partial(
        pl.kernel,
        out_type=jax.ShapeDtypeStruct((E, D), jnp.float32),
        mesh=mesh,
        scratch_types=[
            pltpu.VMEM((per_tile, CH), jnp.int32),
            pltpu.VMEM((NB, CH, D), jnp.float32),
            pltpu.SemaphoreType.DMA((NB,)),
            pltpu.SemaphoreType.DMA((NB,)),
        ],
    )
    def gather_k(xa_hbm, rows_hbm, gx_hbm, idx_v, buf_v, gsem, wsem):
        c = lax.axis_index("c")
        s = lax.axis_index("s")
        w = c * NUM_TEC + s
        chunk0 = w * per_tile
        pltpu.sync_copy(rows_hbm.at[w], idx_v)

        for b in range(NB):
            pltpu.async_copy(xa_hbm.at[idx_v.at[b]], buf_v.at[b], gsem.at[b])

        def group(g, carry):
            for b in range(NB):
                ci = g * NB + b
                pltpu.make_async_copy(
                    xa_hbm.at[idx_v.at[ci]], buf_v.at[b], gsem.at[b]).wait()
                dst = gx_hbm.at[pl.ds((chunk0 + ci) * CH, CH)]
                pltpu.async_copy(buf_v.at[b], dst, wsem.at[b])
                pltpu.make_async_copy(buf_v.at[b], dst, wsem.at[b]).wait()
                nxt = ci + NB

                @pl.when(nxt < per_tile)
                def _():
                    pltpu.async_copy(
                        xa_hbm.at[idx_v.at[nxt]], buf_v.at[b], gsem.at[b])

            return carry

        lax.fori_loop(0, per_tile // NB, group, 0)

    return gather_k


def _make_scatter(N, D, E, NP):
    per_tile = E // (N_TILES * CH)
    rows_nt = NP // NUM_TEC
    wb = 128
    nbs = 2
    mesh = plsc.VectorSubcoreMesh(core_axis_name="c", subcore_axis_name="s")

    @functools.partial(
        pl.kernel,
        out_type=jax.ShapeDtypeStruct((NUM_SC, NP, D), jnp.float32),
        mesh=mesh,
        scratch_types=[
            pltpu.VMEM((per_tile, CH), jnp.int32),
            pltpu.VMEM((nbs, CH, D), jnp.float32),
            pltpu.VMEM_SHARED((NP, D), jnp.float32),
            pltpu.SemaphoreType.DMA((nbs,)),
        ],
    )
    def scatter_k(msg_hbm, cols_hbm, z_hbm, acc_hbm, col_v, buf_v,
                  acc_sh, lsem):
        c = lax.axis_index("c")
        s = lax.axis_index("s")

        def zbody(k, carry):
            pltpu.sync_copy(z_hbm, acc_sh.at[pl.ds(s * rows_nt + k * wb, wb)])
            return carry

        lax.fori_loop(0, rows_nt // wb, zbody, 0)

        w = c * NUM_TEC + s
        chunk0 = w * per_tile
        pltpu.sync_copy(cols_hbm.at[w], col_v)
        plsc.subcore_barrier()

        for b in range(nbs):
            src = msg_hbm.at[pl.ds((chunk0 + b) * CH, CH)]
            pltpu.async_copy(src, buf_v.at[b], lsem.at[b])

        def group(g, carry):
            for b in range(nbs):
                ci = g * nbs + b
                src = msg_hbm.at[pl.ds((chunk0 + ci) * CH, CH)]
                pltpu.make_async_copy(src, buf_v.at[b], lsem.at[b]).wait()
                pltpu.sync_copy(buf_v.at[b], acc_sh.at[col_v.at[ci]], add=True)
                nxt = ci + nbs

                @pl.when(nxt < per_tile)
                def _():
                    pltpu.async_copy(
                        msg_hbm.at[pl.ds((chunk0 + nxt) * CH, CH)],
                        buf_v.at[b], lsem.at[b])

            return carry

        lax.fori_loop(0, per_tile // nbs, group, 0)
        for ci in range((per_tile // nbs) * nbs, per_tile):
            b = ci % nbs
            src = msg_hbm.at[pl.ds((chunk0 + ci) * CH, CH)]
            pltpu.make_async_copy(src, buf_v.at[b], lsem.at[b]).wait()
            pltpu.sync_copy(buf_v.at[b], acc_sh.at[col_v.at[ci]], add=True)
        plsc.subcore_barrier()

        def wbody(k, carry):
            r0 = s * rows_nt + k * wb
            pltpu.sync_copy(acc_sh.at[pl.ds(r0, wb)], acc_hbm.at[c, pl.ds(r0, wb)])
            return carry

        lax.fori_loop(0, rows_nt // wb, wbody, 0)

    return scatter_k


def kernel(x, edge_index, edge_attr, timestamps,
           We1, be1, We2, be2, Wn1, bn1, Wn2, bn2, Wt, bt):
    N, D = x.shape
    E, DE = edge_attr.shape
    K = 5
    Ek = E // K
    per_tile = Ek // (N_TILES * CH)
    NP = 10240
    row = edge_index[0].reshape(K, N_TILES, per_tile, CH)
    col = edge_index[1].reshape(K, N_TILES, per_tile, CH)
    ts2 = timestamps.reshape(E, 1)
    Wn1a = Wn1[:D]
    Wn1b = Wn1[D:]
    H = We1.shape[1]

    BN = 1000
    xa = pl.pallas_call(
        _xa_body,
        grid=(N // BN,),
        in_specs=[
            pl.BlockSpec((BN, D), lambda i: (i, 0)),
            pl.BlockSpec((D, D), lambda i: (0, 0)),
        ],
        out_specs=pl.BlockSpec((BN, D), lambda i: (i, 0)),
        out_shape=jax.ShapeDtypeStruct((N, D), jnp.float32),
    )(x, Wn1a)

    gather_k = _make_gather(N, D, Ek)
    scatter_k = _make_scatter(N, D, Ek, NP)
    zeros = jnp.zeros((128, D), jnp.float32)

    BE = 1280
    full = lambda a: pl.BlockSpec(a.shape, lambda i: tuple(0 for _ in a.shape))
    msg_call = pl.pallas_call(
        _msg_body,
        grid=(Ek // BE,),
        in_specs=[
            pl.BlockSpec((BE, D), lambda i: (i, 0)),
            pl.BlockSpec((BE, DE), lambda i: (i, 0)),
            pl.BlockSpec((BE, 1), lambda i: (i, 0)),
            full(We1), full(be1.reshape(1, H)),
            full(We2), full(be2.reshape(1, D)),
            full(Wt), full(bt.reshape(1, D)),
            full(Wn1b), full(bn1.reshape(1, D)),
            full(Wn2), full(bn2.reshape(1, D)),
        ],
        out_specs=pl.BlockSpec((BE, D), lambda i: (i, 0)),
        out_shape=jax.ShapeDtypeStruct((Ek, D), jnp.float32),
    )

    accs = []
    for k in range(K):
        gx = gather_k(xa, row[k])
        msg = msg_call(
            gx, lax.dynamic_slice_in_dim(edge_attr, k * Ek, Ek),
            lax.dynamic_slice_in_dim(ts2, k * Ek, Ek),
            We1, be1.reshape(1, H), We2, be2.reshape(1, D),
            Wt, bt.reshape(1, D), Wn1b, bn1.reshape(1, D),
            Wn2, bn2.reshape(1, D))
        accs.append(scatter_k(msg, col[k], zeros))

    out = pl.pallas_call(
        _combine_body,
        grid=(N // BN,),
        in_specs=[pl.BlockSpec((NUM_SC, BN, D), lambda i: (0, i, 0))
                  for _ in range(K)]
        + [pl.BlockSpec((BN, D), lambda i: (i, 0))],
        out_specs=pl.BlockSpec((BN, D), lambda i: (i, 0)),
        out_shape=jax.ShapeDtypeStruct((N, D), jnp.float32),
    )(*accs, x)
    return out


# speedup vs baseline: 1.0544x; 1.0544x over previous
import functools

import jax
import jax.numpy as jnp
from jax import lax
from jax.experimental import pallas as pl
from jax.experimental.pallas import tpu as pltpu
from jax.experimental.pallas import tpu_sc as plsc

NUM_SC = 2
NUM_TEC = 16
N_TILES = NUM_SC * NUM_TEC
CH = 80
NB = 5


def _xa_body(x_ref, w_ref, o_ref):
    o_ref[...] = jnp.dot(x_ref[...], w_ref[...], preferred_element_type=jnp.float32)


def _msg_body(gx_ref, ea_ref, ts_ref, We1_ref, be1_ref, We2_ref, be2_ref,
              Wt_ref, bt_ref, Wn1b_ref, bn1_ref, Wn2_ref, bn2_ref, msg_ref):
    h = jnp.maximum(
        jnp.dot(ea_ref[...], We1_ref[...], preferred_element_type=jnp.float32)
        + be1_ref[...], 0.0)
    ef = jnp.dot(h, We2_ref[...], preferred_element_type=jnp.float32) + be2_ref[...]
    z = ts_ref[...] * Wt_ref[...] + bt_ref[...]
    ef = ef * (1.0 / (1.0 + jnp.exp(-z)))
    pre = (gx_ref[...]
           + jnp.dot(ef.astype(jnp.bfloat16), Wn1b_ref[...],
                     preferred_element_type=jnp.float32)
           + bn1_ref[...])
    msg_ref[...] = (jnp.dot(jnp.maximum(pre, 0.0).astype(jnp.bfloat16),
                            Wn2_ref[...],
                            preferred_element_type=jnp.float32) + bn2_ref[...])


def _combine_body(*refs):
    o_ref = refs[-1]
    x_ref = refs[-2]
    tot = x_ref[...]
    for acc_ref in refs[:-2]:
        tot = tot + acc_ref[0] + acc_ref[1]
    o_ref[...] = tot


def _make_gather(N, D, E):
    per_tile = E // (N_TILES * CH)
    mesh = plsc.VectorSubcoreMesh(core_axis_name="c", subcore_axis_name="s")

    @functools.partial(
        pl.kernel,
        out_type=jax.ShapeDtypeStruct((E, D), jnp.float32),
        mesh=mesh,
        scratch_types=[
            pltpu.VMEM((per_tile, CH), jnp.int32),
            pltpu.VMEM((NB, CH, D), jnp.float32),
            pltpu.SemaphoreType.DMA((NB,)),
            pltpu.SemaphoreType.DMA((NB,)),
        ],
    )
    def gather_k(xa_hbm, rows_hbm, gx_hbm, idx_v, buf_v, gsem, wsem):
        c = lax.axis_index("c")
        s = lax.axis_index("s")
        w = c * NUM_TEC + s
        chunk0 = w * per_tile
        pltpu.sync_copy(rows_hbm.at[w], idx_v)

        for b in range(NB):
            pltpu.async_copy(xa_hbm.at[idx_v.at[b]], buf_v.at[b], gsem.at[b])

        def group(g, carry):
            for b in range(NB):
                ci = g * NB + b
                pltpu.make_async_copy(
                    xa_hbm.at[idx_v.at[ci]], buf_v.at[b], gsem.at[b]).wait()
                dst = gx_hbm.at[pl.ds((chunk0 + ci) * CH, CH)]
                pltpu.async_copy(buf_v.at[b], dst, wsem.at[b])
                pltpu.make_async_copy(buf_v.at[b], dst, wsem.at[b]).wait()
                nxt = ci + NB

                @pl.when(nxt < per_tile)
                def _():
                    pltpu.async_copy(
                        xa_hbm.at[idx_v.at[nxt]], buf_v.at[b], gsem.at[b])

            return carry

        lax.fori_loop(0, per_tile // NB, group, 0)

    return gather_k


def _make_scatter(N, D, E, NP):
    per_tile = E // (N_TILES * CH)
    rows_nt = NP // NUM_TEC
    wb = 128
    nbs = 2
    mesh = plsc.VectorSubcoreMesh(core_axis_name="c", subcore_axis_name="s")

    @functools.partial(
        pl.kernel,
        out_type=jax.ShapeDtypeStruct((NUM_SC, NP, D), jnp.float32),
        mesh=mesh,
        scratch_types=[
            pltpu.VMEM((per_tile, CH), jnp.int32),
            pltpu.VMEM((nbs, CH, D), jnp.float32),
            pltpu.VMEM_SHARED((NP, D), jnp.float32),
            pltpu.SemaphoreType.DMA((nbs,)),
        ],
    )
    def scatter_k(msg_hbm, cols_hbm, z_hbm, acc_hbm, col_v, buf_v,
                  acc_sh, lsem):
        c = lax.axis_index("c")
        s = lax.axis_index("s")

        def zbody(k, carry):
            pltpu.sync_copy(z_hbm, acc_sh.at[pl.ds(s * rows_nt + k * wb, wb)])
            return carry

        lax.fori_loop(0, rows_nt // wb, zbody, 0)

        w = c * NUM_TEC + s
        chunk0 = w * per_tile
        pltpu.sync_copy(cols_hbm.at[w], col_v)
        plsc.subcore_barrier()

        for b in range(nbs):
            src = msg_hbm.at[pl.ds((chunk0 + b) * CH, CH)]
            pltpu.async_copy(src, buf_v.at[b], lsem.at[b])

        def group(g, carry):
            for b in range(nbs):
                ci = g * nbs + b
                src = msg_hbm.at[pl.ds((chunk0 + ci) * CH, CH)]
                pltpu.make_async_copy(src, buf_v.at[b], lsem.at[b]).wait()
                pltpu.sync_copy(buf_v.at[b], acc_sh.at[col_v.at[ci]], add=True)
                nxt = ci + nbs

                @pl.when(nxt < per_tile)
                def _():
                    pltpu.async_copy(
                        msg_hbm.at[pl.ds((chunk0 + nxt) * CH, CH)],
                        buf_v.at[b], lsem.at[b])

            return carry

        lax.fori_loop(0, per_tile // nbs, group, 0)
        for ci in range((per_tile // nbs) * nbs, per_tile):
            b = ci % nbs
            src = msg_hbm.at[pl.ds((chunk0 + ci) * CH, CH)]
            pltpu.make_async_copy(src, buf_v.at[b], lsem.at[b]).wait()
            pltpu.sync_copy(buf_v.at[b], acc_sh.at[col_v.at[ci]], add=True)
        plsc.subcore_barrier()

        def wbody(k, carry):
            r0 = s * rows_nt + k * wb
            pltpu.sync_copy(acc_sh.at[pl.ds(r0, wb)], acc_hbm.at[c, pl.ds(r0, wb)])
            return carry

        lax.fori_loop(0, rows_nt // wb, wbody, 0)

    return scatter_k


def kernel(x, edge_index, edge_attr, timestamps,
           We1, be1, We2, be2, Wn1, bn1, Wn2, bn2, Wt, bt):
    N, D = x.shape
    E, DE = edge_attr.shape
    K = 1
    Ek = E // K
    per_tile = Ek // (N_TILES * CH)
    NP = 10240
    row = edge_index[0].reshape(K, N_TILES, per_tile, CH)
    col = edge_index[1].reshape(K, N_TILES, per_tile, CH)
    ts2 = timestamps.reshape(E, 1)
    Wn1a = Wn1[:D]
    Wn1b = Wn1[D:].astype(jnp.bfloat16)
    Wn2b = Wn2.astype(jnp.bfloat16)
    H = We1.shape[1]

    BN = 1000
    xa = pl.pallas_call(
        _xa_body,
        grid=(N // BN,),
        in_specs=[
            pl.BlockSpec((BN, D), lambda i: (i, 0)),
            pl.BlockSpec((D, D), lambda i: (0, 0)),
        ],
        out_specs=pl.BlockSpec((BN, D), lambda i: (i, 0)),
        out_shape=jax.ShapeDtypeStruct((N, D), jnp.float32),
    )(x, Wn1a)

    gather_k = _make_gather(N, D, Ek)
    scatter_k = _make_scatter(N, D, Ek, NP)
    zeros = jnp.zeros((128, D), jnp.float32)

    BE = 1280
    full = lambda a: pl.BlockSpec(a.shape, lambda i: tuple(0 for _ in a.shape))
    msg_call = pl.pallas_call(
        _msg_body,
        grid=(Ek // BE,),
        in_specs=[
            pl.BlockSpec((BE, D), lambda i: (i, 0)),
            pl.BlockSpec((BE, DE), lambda i: (i, 0)),
            pl.BlockSpec((BE, 1), lambda i: (i, 0)),
            full(We1), full(be1.reshape(1, H)),
            full(We2), full(be2.reshape(1, D)),
            full(Wt), full(bt.reshape(1, D)),
            full(Wn1b), full(bn1.reshape(1, D)),
            full(Wn2), full(bn2.reshape(1, D)),
        ],
        out_specs=pl.BlockSpec((BE, D), lambda i: (i, 0)),
        out_shape=jax.ShapeDtypeStruct((Ek, D), jnp.float32),
    )

    accs = []
    for k in range(K):
        gx = gather_k(xa, row[k])
        msg = msg_call(
            gx, lax.dynamic_slice_in_dim(edge_attr, k * Ek, Ek),
            lax.dynamic_slice_in_dim(ts2, k * Ek, Ek),
            We1, be1.reshape(1, H), We2, be2.reshape(1, D),
            Wt, bt.reshape(1, D), Wn1b, bn1.reshape(1, D),
            Wn2b, bn2.reshape(1, D))
        accs.append(scatter_k(msg, col[k], zeros))

    out = pl.pallas_call(
        _combine_body,
        grid=(N // BN,),
        in_specs=[pl.BlockSpec((NUM_SC, BN, D), lambda i: (0, i, 0))
                  for _ in range(K)]
        + [pl.BlockSpec((BN, D), lambda i: (i, 0))],
        out_specs=pl.BlockSpec((BN, D), lambda i: (i, 0)),
        out_shape=jax.ShapeDtypeStruct((N, D), jnp.float32),
    )(*accs, x)
    return out


# speedup vs baseline: 1.0732x; 1.0178x over previous
import functools

import jax
import jax.numpy as jnp
from jax import lax
from jax.experimental import pallas as pl
from jax.experimental.pallas import tpu as pltpu
from jax.experimental.pallas import tpu_sc as plsc

NUM_SC = 2
NUM_TEC = 16
N_TILES = NUM_SC * NUM_TEC
CH = 40
NB = 2


def _xa_body(x_ref, w_ref, o_ref):
    o_ref[...] = jnp.dot(x_ref[...], w_ref[...], preferred_element_type=jnp.float32)


def _msg_body(gx_ref, ea_ref, ts_ref, We1_ref, be1_ref, We2_ref, be2_ref,
              Wt_ref, bt_ref, Wn1b_ref, bn1_ref, Wn2_ref, bn2_ref, msg_ref):
    h = jnp.maximum(
        jnp.dot(ea_ref[...], We1_ref[...], preferred_element_type=jnp.float32)
        + be1_ref[...], 0.0)
    ef = jnp.dot(h, We2_ref[...], preferred_element_type=jnp.float32) + be2_ref[...]
    z = ts_ref[...] * Wt_ref[...] + bt_ref[...]
    ef = ef * (1.0 / (1.0 + jnp.exp(-z)))
    pre = (gx_ref[...]
           + jnp.dot(ef.astype(jnp.bfloat16), Wn1b_ref[...],
                     preferred_element_type=jnp.float32)
           + bn1_ref[...])
    msg_ref[...] = (jnp.dot(jnp.maximum(pre, 0.0).astype(jnp.bfloat16),
                            Wn2_ref[...],
                            preferred_element_type=jnp.float32) + bn2_ref[...])


def _combine_body(*refs):
    o_ref = refs[-1]
    x_ref = refs[-2]
    tot = x_ref[...]
    for acc_ref in refs[:-2]:
        tot = tot + acc_ref[0] + acc_ref[1]
    o_ref[...] = tot


def _make_gather(N, D, E):
    per_tile = E // (N_TILES * CH)
    NP2 = 10240
    rows_st = NP2 // NUM_TEC
    mesh = plsc.VectorSubcoreMesh(core_axis_name="c", subcore_axis_name="s")

    @functools.partial(
        pl.kernel,
        out_type=jax.ShapeDtypeStruct((E, D), jnp.float32),
        mesh=mesh,
        scratch_types=[
            pltpu.VMEM((per_tile, CH), jnp.int32),
            pltpu.VMEM((NB, CH, D), jnp.float32),
            pltpu.VMEM_SHARED((NP2, D), jnp.float32),
            pltpu.SemaphoreType.DMA((NB,)),
            pltpu.SemaphoreType.DMA((NB,)),
        ],
    )
    def gather_k(xa_hbm, rows_hbm, gx_hbm, idx_v, buf_v, xa_sh, gsem, wsem):
        c = lax.axis_index("c")
        s = lax.axis_index("s")
        w = c * NUM_TEC + s
        chunk0 = w * per_tile

        r0 = s * rows_st
        last_full = N // rows_st
        rem = N - last_full * rows_st

        @pl.when(s < last_full)
        def _():
            pltpu.sync_copy(xa_hbm.at[pl.ds(r0, rows_st)],
                            xa_sh.at[pl.ds(r0, rows_st)])

        if rem:
            @pl.when(s == last_full)
            def _():
                pltpu.sync_copy(xa_hbm.at[pl.ds(last_full * rows_st, rem)],
                                xa_sh.at[pl.ds(last_full * rows_st, rem)])

        pltpu.sync_copy(rows_hbm.at[w], idx_v)
        plsc.subcore_barrier()

        for b in range(NB):
            pltpu.async_copy(xa_sh.at[idx_v.at[b]], buf_v.at[b], gsem.at[b])

        def group(g, carry):
            for b in range(NB):
                ci = g * NB + b
                pltpu.make_async_copy(
                    xa_sh.at[idx_v.at[ci]], buf_v.at[b], gsem.at[b]).wait()
                dst = gx_hbm.at[pl.ds((chunk0 + ci) * CH, CH)]
                pltpu.async_copy(buf_v.at[b], dst, wsem.at[b])
                pltpu.make_async_copy(buf_v.at[b], dst, wsem.at[b]).wait()
                nxt = ci + NB

                @pl.when(nxt < per_tile)
                def _():
                    pltpu.async_copy(
                        xa_sh.at[idx_v.at[nxt]], buf_v.at[b], gsem.at[b])

            return carry

        lax.fori_loop(0, per_tile // NB, group, 0)

    return gather_k


def _make_scatter(N, D, E, NP):
    per_tile = E // (N_TILES * CH)
    rows_nt = NP // NUM_TEC
    wb = 128
    nbs = 2
    mesh = plsc.VectorSubcoreMesh(core_axis_name="c", subcore_axis_name="s")

    @functools.partial(
        pl.kernel,
        out_type=jax.ShapeDtypeStruct((NUM_SC, NP, D), jnp.float32),
        mesh=mesh,
        scratch_types=[
            pltpu.VMEM((per_tile, CH), jnp.int32),
            pltpu.VMEM((nbs, CH, D), jnp.float32),
            pltpu.VMEM_SHARED((NP, D), jnp.float32),
            pltpu.SemaphoreType.DMA((nbs,)),
        ],
    )
    def scatter_k(msg_hbm, cols_hbm, z_hbm, acc_hbm, col_v, buf_v,
                  acc_sh, lsem):
        c = lax.axis_index("c")
        s = lax.axis_index("s")

        def zbody(k, carry):
            pltpu.sync_copy(z_hbm, acc_sh.at[pl.ds(s * rows_nt + k * wb, wb)])
            return carry

        lax.fori_loop(0, rows_nt // wb, zbody, 0)

        w = c * NUM_TEC + s
        chunk0 = w * per_tile
        pltpu.sync_copy(cols_hbm.at[w], col_v)
        plsc.subcore_barrier()

        for b in range(nbs):
            src = msg_hbm.at[pl.ds((chunk0 + b) * CH, CH)]
            pltpu.async_copy(src, buf_v.at[b], lsem.at[b])

        def group(g, carry):
            for b in range(nbs):
                ci = g * nbs + b
                src = msg_hbm.at[pl.ds((chunk0 + ci) * CH, CH)]
                pltpu.make_async_copy(src, buf_v.at[b], lsem.at[b]).wait()
                pltpu.sync_copy(buf_v.at[b], acc_sh.at[col_v.at[ci]], add=True)
                nxt = ci + nbs

                @pl.when(nxt < per_tile)
                def _():
                    pltpu.async_copy(
                        msg_hbm.at[pl.ds((chunk0 + nxt) * CH, CH)],
                        buf_v.at[b], lsem.at[b])

            return carry

        lax.fori_loop(0, per_tile // nbs, group, 0)
        for ci in range((per_tile // nbs) * nbs, per_tile):
            b = ci % nbs
            src = msg_hbm.at[pl.ds((chunk0 + ci) * CH, CH)]
            pltpu.make_async_copy(src, buf_v.at[b], lsem.at[b]).wait()
            pltpu.sync_copy(buf_v.at[b], acc_sh.at[col_v.at[ci]], add=True)
        plsc.subcore_barrier()

        def wbody(k, carry):
            r0 = s * rows_nt + k * wb
            pltpu.sync_copy(acc_sh.at[pl.ds(r0, wb)], acc_hbm.at[c, pl.ds(r0, wb)])
            return carry

        lax.fori_loop(0, rows_nt // wb, wbody, 0)

    return scatter_k


def kernel(x, edge_index, edge_attr, timestamps,
           We1, be1, We2, be2, Wn1, bn1, Wn2, bn2, Wt, bt):
    N, D = x.shape
    E, DE = edge_attr.shape
    K = 1
    Ek = E // K
    per_tile = Ek // (N_TILES * CH)
    NP = 10240
    row = edge_index[0].reshape(K, N_TILES, per_tile, CH)
    col = edge_index[1].reshape(K, N_TILES, per_tile, CH)
    ts2 = timestamps.reshape(E, 1)
    Wn1a = Wn1[:D]
    Wn1b = Wn1[D:].astype(jnp.bfloat16)
    Wn2b = Wn2.astype(jnp.bfloat16)
    H = We1.shape[1]

    BN = 1000
    xa = pl.pallas_call(
        _xa_body,
        grid=(N // BN,),
        in_specs=[
            pl.BlockSpec((BN, D), lambda i: (i, 0)),
            pl.BlockSpec((D, D), lambda i: (0, 0)),
        ],
        out_specs=pl.BlockSpec((BN, D), lambda i: (i, 0)),
        out_shape=jax.ShapeDtypeStruct((N, D), jnp.float32),
    )(x, Wn1a)

    gather_k = _make_gather(N, D, Ek)
    scatter_k = _make_scatter(N, D, Ek, NP)
    zeros = jnp.zeros((128, D), jnp.float32)

    BE = 1280
    full = lambda a: pl.BlockSpec(a.shape, lambda i: tuple(0 for _ in a.shape))
    msg_call = pl.pallas_call(
        _msg_body,
        grid=(Ek // BE,),
        in_specs=[
            pl.BlockSpec((BE, D), lambda i: (i, 0)),
            pl.BlockSpec((BE, DE), lambda i: (i, 0)),
            pl.BlockSpec((BE, 1), lambda i: (i, 0)),
            full(We1), full(be1.reshape(1, H)),
            full(We2), full(be2.reshape(1, D)),
            full(Wt), full(bt.reshape(1, D)),
            full(Wn1b), full(bn1.reshape(1, D)),
            full(Wn2), full(bn2.reshape(1, D)),
        ],
        out_specs=pl.BlockSpec((BE, D), lambda i: (i, 0)),
        out_shape=jax.ShapeDtypeStruct((Ek, D), jnp.float32),
    )

    accs = []
    for k in range(K):
        gx = gather_k(xa, row[k])
        msg = msg_call(
            gx, lax.dynamic_slice_in_dim(edge_attr, k * Ek, Ek),
            lax.dynamic_slice_in_dim(ts2, k * Ek, Ek),
            We1, be1.reshape(1, H), We2, be2.reshape(1, D),
            Wt, bt.reshape(1, D), Wn1b, bn1.reshape(1, D),
            Wn2b, bn2.reshape(1, D))
        accs.append(scatter_k(msg, col[k], zeros))

    out = pl.pallas_call(
        _combine_body,
        grid=(N // BN,),
        in_specs=[pl.BlockSpec((NUM_SC, BN, D), lambda i: (0, i, 0))
                  for _ in range(K)]
        + [pl.BlockSpec((BN, D), lambda i: (i, 0))],
        out_specs=pl.BlockSpec((BN, D), lambda i: (i, 0)),
        out_shape=jax.ShapeDtypeStruct((N, D), jnp.float32),
    )(*accs, x)
    return out
